# baseline (speedup 1.0000x reference)
import functools

import jax
import jax.numpy as jnp
from jax import lax
from jax.experimental import pallas as pl
from jax.experimental.pallas import tpu as pltpu
from jax.experimental.pallas import tpu_sc as plsc

N = 10000
E = 320000
D = 128
B = 8192
NC = 2
NS = 16
CH = 80
NCHUNK = 128
EP = NC * NS * NCHUNK * CH
NJUNK = 8
RPT = 624
RPT_LAST = N - (NS - 1) * RPT
ZR = 16
BPT = B // (NC * NS)
GCH = 128

_sc_mesh = plsc.VectorSubcoreMesh(
    core_axis_name="c", subcore_axis_name="s", num_cores=NC, num_subcores=NS
)


@functools.partial(
    pl.kernel,
    out_type=jax.ShapeDtypeStruct((NC, N, D), jnp.float32),
    mesh=_sc_mesh,
    scratch_types=[
        pltpu.VMEM((NCHUNK // 2, CH), jnp.int32),
        pltpu.VMEM((NCHUNK // 2, CH), jnp.int32),
        pltpu.VMEM((3, CH, D), jnp.float32),
        pltpu.VMEM((ZR, D), jnp.float32),
        pltpu.VMEM_SHARED((N + NJUNK, D), jnp.float32),
        pltpu.SemaphoreType.DMA,
        pltpu.SemaphoreType.DMA,
    ],
)
def _neighbor_sum(x_hbm, row2_hbm, col2_hbm, out_hbm,
                  row_v, col_v, rows_v, zb, nb_s, sem0, sem1):
    c = lax.axis_index("c")
    s = lax.axis_index("s")
    tg = c * NS + s

    zeros16 = jnp.zeros((16,), jnp.float32)
    for i in range(ZR):
        for k in range(D // 16):
            zb[i, pl.ds(k * 16, 16)] = zeros16
    for q in range(RPT // ZR):
        pltpu.sync_copy(zb, nb_s.at[pl.ds(s * RPT + q * ZR, ZR)])

    @pl.when(s == NS - 1)
    def _zero_tail():
        pltpu.sync_copy(zb, nb_s.at[pl.ds(NS * RPT, N - NS * RPT)])

    plsc.subcore_barrier()

    HN = NCHUNK // 2
    for h in range(2):
        hbase = tg * NCHUNK + h * HN
        pltpu.sync_copy(row2_hbm.at[pl.ds(hbase, HN)], row_v)
        pltpu.sync_copy(col2_hbm.at[pl.ds(hbase, HN)], col_v)
        pltpu.async_copy(x_hbm.at[col_v.at[0]], rows_v.at[0], sem0).wait()

        def body(g, carry):
            j0 = 2 * g
            b0 = j0 % 3
            b1 = (j0 + 1) % 3
            b2 = (j0 + 2) % 3
            d1 = pltpu.async_copy(x_hbm.at[col_v.at[j0 + 1]], rows_v.at[b1],
                                  sem1)
            d2 = pltpu.async_copy(x_hbm.at[col_v.at[j0 + 2]], rows_v.at[b2],
                                  sem0)
            pltpu.sync_copy(rows_v.at[b0], nb_s.at[row_v.at[j0]], add=True)
            d1.wait()
            pltpu.sync_copy(rows_v.at[b1], nb_s.at[row_v.at[j0 + 1]],
                            add=True)
            d2.wait()
            return carry

        lax.fori_loop(0, (HN - 2) // 2, body, 0)

        e0, e1 = HN - 2, HN - 1
        d = pltpu.async_copy(x_hbm.at[col_v.at[e1]], rows_v.at[e1 % 3], sem1)
        pltpu.sync_copy(rows_v.at[e0 % 3], nb_s.at[row_v.at[e0]], add=True)
        d.wait()
        pltpu.sync_copy(rows_v.at[e1 % 3], nb_s.at[row_v.at[e1]], add=True)

    plsc.subcore_barrier()

    pltpu.sync_copy(nb_s.at[pl.ds(s * RPT, RPT)],
                    out_hbm.at[c, pl.ds(s * RPT, RPT)])

    @pl.when(s == NS - 1)
    def _write_tail():
        pltpu.sync_copy(nb_s.at[pl.ds(NS * RPT, N - NS * RPT)],
                        out_hbm.at[c, pl.ds(NS * RPT, N - NS * RPT)])


@functools.partial(
    pl.kernel,
    out_type=[jax.ShapeDtypeStruct((B, D), jnp.float32)] * 7,
    mesh=_sc_mesh,
    scratch_types=[
        pltpu.VMEM((GCH,), jnp.int32),
        pltpu.VMEM((GCH, D), jnp.float32),
        pltpu.SemaphoreType.DMA,
    ],
)
def _gather7(x1, x2, x3, re_hbm, hid, tid, rid,
             oh1, oh2, oh3, ot1, ot2, ot3, org,
             idx_v, buf, sem):
    c = lax.axis_index("c")
    s = lax.axis_index("s")
    w = s * NC + c

    for k in range(BPT // GCH):
        base = w * BPT + k * GCH
        pltpu.sync_copy(hid.at[pl.ds(base, GCH)], idx_v)
        for src, dst in ((x1, oh1), (x2, oh2), (x3, oh3)):
            pltpu.async_copy(src.at[idx_v], buf, sem).wait()
            pltpu.sync_copy(buf, dst.at[pl.ds(base, GCH)])
        pltpu.sync_copy(tid.at[pl.ds(base, GCH)], idx_v)
        for src, dst in ((x1, ot1), (x2, ot2), (x3, ot3)):
            pltpu.async_copy(src.at[idx_v], buf, sem).wait()
            pltpu.sync_copy(buf, dst.at[pl.ds(base, GCH)])
        pltpu.sync_copy(rid.at[pl.ds(base, GCH)], idx_v)
        pltpu.async_copy(re_hbm.at[idx_v], buf, sem).wait()
        pltpu.sync_copy(buf, org.at[pl.ds(base, GCH)])


BR = 1000
NB = N // BR


def _mlp_body(x_ref, nb_ref, W1_ref, b1_ref, g_ref, be_ref, W2_ref, b2_ref,
              eps_ref, out_ref):
    dot = lambda a, b: jnp.dot(a, b, preferred_element_type=jnp.float32)
    h0 = x_ref[...] * eps_ref[0, 0] + nb_ref[0] + nb_ref[1]
    hb = dot(h0, W1_ref[...]) + b1_ref[...]
    mu = jnp.mean(hb, axis=0, keepdims=True)
    dm = hb - mu
    var = jnp.mean(dm * dm, axis=0, keepdims=True)
    hn = dm * lax.rsqrt(var + 1e-5) * g_ref[...] + be_ref[...]
    out_ref[...] = dot(jnp.maximum(hn, 0.0), W2_ref[...]) + b2_ref[...]


def _mlp(x, nb, W1, b1, g, be, W2, b2, opeps):
    vm = pl.BlockSpec(memory_space=pltpu.VMEM)
    return pl.pallas_call(
        _mlp_body,
        in_specs=[vm] * 8 + [pl.BlockSpec(memory_space=pltpu.SMEM)],
        out_specs=vm,
        out_shape=jax.ShapeDtypeStruct((N, D), jnp.float32),
    )(x, nb, W1, b1, g, be, W2, b2, opeps)


SB = 1024
NSB = B // SB


def _score_body(h1, h2, h3, t1, t2, t3, rg, J1, J2, J3, jkb, Wh, Wr, Wt,
                b1r, w2r, b2s, out_ref):
    dot = lambda a, b: jnp.dot(a, b, preferred_element_type=jnp.float32)
    A1 = dot(J1[...], Wh[...])
    A2 = dot(J2[...], Wh[...])
    A3 = dot(J3[...], Wh[...])
    C1 = dot(J1[...], Wt[...])
    C2 = dot(J2[...], Wt[...])
    C3 = dot(J3[...], Wt[...])
    bias = dot(jkb[...], Wh[...]) + dot(jkb[...], Wt[...]) + b1r[...]
    pre = (dot(h1[...], A1) + dot(h2[...], A2) + dot(h3[...], A3)
           + dot(t1[...], C1) + dot(t2[...], C2) + dot(t3[...], C3)
           + dot(rg[...], Wr[...]) + bias)
    hr = jnp.maximum(pre, 0.0)
    out_ref[...] = jnp.sum(hr * w2r[...], axis=1) + b2s[0, 0]


def _score(h1, h2, h3, t1, t2, t3, rg, J1, J2, J3, jkb, Wh, Wr, Wt,
           b1r, w2r, b2s):
    blk = pl.BlockSpec((SB, D), lambda i: (i, 0))
    full = lambda shape: pl.BlockSpec(shape, lambda i: (0,) * len(shape))
    return pl.pallas_call(
        _score_body,
        grid=(NSB,),
        in_specs=[blk] * 7 + [full((D, D))] * 3 + [full((1, D))]
        + [full((D, D))] * 3 + [full((1, D)), full((1, D))]
        + [pl.BlockSpec(memory_space=pltpu.SMEM)],
        out_specs=pl.BlockSpec((SB,), lambda i: (i,)),
        out_shape=jax.ShapeDtypeStruct((B,), jnp.float32),
    )(h1, h2, h3, t1, t2, t3, rg, J1, J2, J3, jkb, Wh, Wr, Wt, b1r, w2r, b2s)


def kernel(edge_index, head_ids, relation_ids, tail_ids, ent_emb, rel_emb,
           W1_0, b1_0, g_0, be_0, W2_0, b2_0, eps_0,
           W1_1, b1_1, g_1, be_1, W2_1, b2_1, eps_1,
           W1_2, b1_2, g_2, be_2, W2_2, b2_2, eps_2,
           jk_W, jk_b, lp_W1, lp_b1, lp_W2, lp_b2):
    pad_r = jnp.full((EP - E,), N, jnp.int32)
    pad_c = jnp.zeros((EP - E,), jnp.int32)
    row2 = jnp.concatenate([edge_index[0], pad_r]).reshape(EP // CH, CH)
    col2 = jnp.concatenate([edge_index[1], pad_c]).reshape(EP // CH, CH)

    params = [
        (W1_0, b1_0, g_0, be_0, W2_0, b2_0, eps_0),
        (W1_1, b1_1, g_1, be_1, W2_1, b2_1, eps_1),
        (W1_2, b1_2, g_2, be_2, W2_2, b2_2, eps_2),
    ]
    x = ent_emb
    outs = []
    for (W1, b1, g, be, W2, b2, eps) in params:
        nb = _neighbor_sum(x, row2, col2)
        x = _mlp(x, nb, W1, b1.reshape(1, D), g.reshape(1, D),
                 be.reshape(1, D), W2, b2.reshape(1, D),
                 (1.0 + eps).reshape(1, 1))
        outs.append(x)

    x1, x2, x3 = outs
    h1, h2, h3, t1, t2, t3, rg = _gather7(
        x1, x2, x3, rel_emb, head_ids, tail_ids, relation_ids)

    J1, J2, J3 = jk_W[0:D], jk_W[D:2 * D], jk_W[2 * D:3 * D]
    Wh, Wr, Wt = lp_W1[0:D], lp_W1[D:2 * D], lp_W1[2 * D:3 * D]
    return _score(h1, h2, h3, t1, t2, t3, rg, J1, J2, J3,
                  jk_b.reshape(1, D), Wh, Wr, Wt,
                  lp_b1.reshape(1, D), lp_W2.reshape(1, D),
                  lp_b2.reshape(1, 1))


# speedup vs baseline: 3.2898x; 3.2898x over previous
import functools

import jax
import jax.numpy as jnp
from jax import lax
from jax.experimental import pallas as pl
from jax.experimental.pallas import tpu as pltpu
from jax.experimental.pallas import tpu_sc as plsc

N = 10000
E = 320000
D = 128
B = 8192
NC = 2
NS = 16
CH = 125
NCHUNK = 80
RPT = 624
RPT_LAST = N - (NS - 1) * RPT
ZR = 16
BPT = B // (NC * NS)
GCH = 128

_sc_mesh = plsc.VectorSubcoreMesh(
    core_axis_name="c", subcore_axis_name="s", num_cores=NC, num_subcores=NS
)


@functools.partial(
    pl.kernel,
    out_type=jax.ShapeDtypeStruct((NC, N, D), jnp.float32),
    mesh=_sc_mesh,
    scratch_types=[
        pltpu.VMEM((NCHUNK // 2, CH), jnp.int32),
        pltpu.VMEM((NCHUNK // 2, CH), jnp.int32),
        pltpu.VMEM((2, CH, D), jnp.float32),
        pltpu.VMEM((ZR, D), jnp.float32),
        pltpu.VMEM_SHARED((N, D), jnp.float32),
        pltpu.SemaphoreType.DMA,
    ],
)
def _neighbor_sum(x_hbm, row2_hbm, col2_hbm, out_hbm,
                  row_v, col_v, rows_v, zb, nb_s, sem):
    c = lax.axis_index("c")
    s = lax.axis_index("s")
    tg = c * NS + s

    zeros16 = jnp.zeros((16,), jnp.float32)
    for i in range(ZR):
        for k in range(D // 16):
            zb[i, pl.ds(k * 16, 16)] = zeros16
    for q in range(RPT // ZR):
        pltpu.sync_copy(zb, nb_s.at[pl.ds(s * RPT + q * ZR, ZR)])

    @pl.when(s == NS - 1)
    def _zero_tail():
        pltpu.sync_copy(zb, nb_s.at[pl.ds(NS * RPT, N - NS * RPT)])

    plsc.subcore_barrier()

    HN = NCHUNK // 2
    for h in range(2):
        hbase = tg * NCHUNK + h * HN
        pltpu.sync_copy(row2_hbm.at[pl.ds(hbase, HN)], row_v)
        pltpu.sync_copy(col2_hbm.at[pl.ds(hbase, HN)], col_v)
        pltpu.async_copy(x_hbm.at[col_v.at[0]], rows_v.at[0], sem).wait()

        def body(j, carry):
            b = j % 2

            @pl.when(j + 1 < HN)
            def _steady():
                d = pltpu.async_copy(x_hbm.at[col_v.at[j + 1]],
                                     rows_v.at[1 - b], sem)
                pltpu.sync_copy(rows_v.at[b], nb_s.at[row_v.at[j]], add=True)
                d.wait()

            @pl.when(j + 1 >= HN)
            def _last():
                pltpu.sync_copy(rows_v.at[b], nb_s.at[row_v.at[j]], add=True)
            return carry

        lax.fori_loop(0, HN, body, 0)

    plsc.subcore_barrier()

    pltpu.sync_copy(nb_s.at[pl.ds(s * RPT, RPT)],
                    out_hbm.at[c, pl.ds(s * RPT, RPT)])

    @pl.when(s == NS - 1)
    def _write_tail():
        pltpu.sync_copy(nb_s.at[pl.ds(NS * RPT, N - NS * RPT)],
                        out_hbm.at[c, pl.ds(NS * RPT, N - NS * RPT)])


@functools.partial(
    pl.kernel,
    out_type=[jax.ShapeDtypeStruct((B, D), jnp.float32)] * 7,
    mesh=_sc_mesh,
    scratch_types=[
        pltpu.VMEM((GCH,), jnp.int32),
        pltpu.VMEM((GCH, D), jnp.float32),
        pltpu.SemaphoreType.DMA,
    ],
)
def _gather7(x1, x2, x3, re_hbm, hid, tid, rid,
             oh1, oh2, oh3, ot1, ot2, ot3, org,
             idx_v, buf, sem):
    c = lax.axis_index("c")
    s = lax.axis_index("s")
    w = s * NC + c

    for k in range(BPT // GCH):
        base = w * BPT + k * GCH
        pltpu.sync_copy(hid.at[pl.ds(base, GCH)], idx_v)
        for src, dst in ((x1, oh1), (x2, oh2), (x3, oh3)):
            pltpu.async_copy(src.at[idx_v], buf, sem).wait()
            pltpu.sync_copy(buf, dst.at[pl.ds(base, GCH)])
        pltpu.sync_copy(tid.at[pl.ds(base, GCH)], idx_v)
        for src, dst in ((x1, ot1), (x2, ot2), (x3, ot3)):
            pltpu.async_copy(src.at[idx_v], buf, sem).wait()
            pltpu.sync_copy(buf, dst.at[pl.ds(base, GCH)])
        pltpu.sync_copy(rid.at[pl.ds(base, GCH)], idx_v)
        pltpu.async_copy(re_hbm.at[idx_v], buf, sem).wait()
        pltpu.sync_copy(buf, org.at[pl.ds(base, GCH)])


BR = 1000
NB = N // BR


def _mlp_body(x_ref, nb_ref, W1_ref, b1_ref, g_ref, be_ref, W2_ref, b2_ref,
              eps_ref, out_ref):
    dot = lambda a, b: jnp.dot(a, b, preferred_element_type=jnp.float32)
    h0 = x_ref[...] * eps_ref[0, 0] + nb_ref[0] + nb_ref[1]
    hb = dot(h0, W1_ref[...]) + b1_ref[...]
    mu = jnp.mean(hb, axis=0, keepdims=True)
    dm = hb - mu
    var = jnp.mean(dm * dm, axis=0, keepdims=True)
    hn = dm * lax.rsqrt(var + 1e-5) * g_ref[...] + be_ref[...]
    out_ref[...] = dot(jnp.maximum(hn, 0.0), W2_ref[...]) + b2_ref[...]


def _mlp(x, nb, W1, b1, g, be, W2, b2, opeps):
    vm = pl.BlockSpec(memory_space=pltpu.VMEM)
    return pl.pallas_call(
        _mlp_body,
        in_specs=[vm] * 8 + [pl.BlockSpec(memory_space=pltpu.SMEM)],
        out_specs=vm,
        out_shape=jax.ShapeDtypeStruct((N, D), jnp.float32),
    )(x, nb, W1, b1, g, be, W2, b2, opeps)


SB = 1024
NSB = B // SB


def _score_body(h1, h2, h3, t1, t2, t3, rg, J1, J2, J3, jkb, Wh, Wr, Wt,
                b1r, w2r, b2s, out_ref):
    dot = lambda a, b: jnp.dot(a, b, preferred_element_type=jnp.float32)
    A1 = dot(J1[...], Wh[...])
    A2 = dot(J2[...], Wh[...])
    A3 = dot(J3[...], Wh[...])
    C1 = dot(J1[...], Wt[...])
    C2 = dot(J2[...], Wt[...])
    C3 = dot(J3[...], Wt[...])
    bias = dot(jkb[...], Wh[...]) + dot(jkb[...], Wt[...]) + b1r[...]
    pre = (dot(h1[...], A1) + dot(h2[...], A2) + dot(h3[...], A3)
           + dot(t1[...], C1) + dot(t2[...], C2) + dot(t3[...], C3)
           + dot(rg[...], Wr[...]) + bias)
    hr = jnp.maximum(pre, 0.0)
    out_ref[...] = jnp.sum(hr * w2r[...], axis=1) + b2s[0, 0]


def _score(h1, h2, h3, t1, t2, t3, rg, J1, J2, J3, jkb, Wh, Wr, Wt,
           b1r, w2r, b2s):
    blk = pl.BlockSpec((SB, D), lambda i: (i, 0))
    full = lambda shape: pl.BlockSpec(shape, lambda i: (0,) * len(shape))
    return pl.pallas_call(
        _score_body,
        grid=(NSB,),
        in_specs=[blk] * 7 + [full((D, D))] * 3 + [full((1, D))]
        + [full((D, D))] * 3 + [full((1, D)), full((1, D))]
        + [pl.BlockSpec(memory_space=pltpu.SMEM)],
        out_specs=pl.BlockSpec((SB,), lambda i: (i,)),
        out_shape=jax.ShapeDtypeStruct((B,), jnp.float32),
    )(h1, h2, h3, t1, t2, t3, rg, J1, J2, J3, jkb, Wh, Wr, Wt, b1r, w2r, b2s)


def kernel(edge_index, head_ids, relation_ids, tail_ids, ent_emb, rel_emb,
           W1_0, b1_0, g_0, be_0, W2_0, b2_0, eps_0,
           W1_1, b1_1, g_1, be_1, W2_1, b2_1, eps_1,
           W1_2, b1_2, g_2, be_2, W2_2, b2_2, eps_2,
           jk_W, jk_b, lp_W1, lp_b1, lp_W2, lp_b2):
    row2 = edge_index[0].reshape(E // CH, CH)
    col2 = edge_index[1].reshape(E // CH, CH)

    params = [
        (W1_0, b1_0, g_0, be_0, W2_0, b2_0, eps_0),
        (W1_1, b1_1, g_1, be_1, W2_1, b2_1, eps_1),
        (W1_2, b1_2, g_2, be_2, W2_2, b2_2, eps_2),
    ]
    x = ent_emb
    outs = []
    for (W1, b1, g, be, W2, b2, eps) in params:
        nb = _neighbor_sum(x, row2, col2)
        x = _mlp(x, nb, W1, b1.reshape(1, D), g.reshape(1, D),
                 be.reshape(1, D), W2, b2.reshape(1, D),
                 (1.0 + eps).reshape(1, 1))
        outs.append(x)

    x1, x2, x3 = outs
    h1, h2, h3, t1, t2, t3, rg = _gather7(
        x1, x2, x3, rel_emb, head_ids, tail_ids, relation_ids)

    J1, J2, J3 = jk_W[0:D], jk_W[D:2 * D], jk_W[2 * D:3 * D]
    Wh, Wr, Wt = lp_W1[0:D], lp_W1[D:2 * D], lp_W1[2 * D:3 * D]
    return _score(h1, h2, h3, t1, t2, t3, rg, J1, J2, J3,
                  jk_b.reshape(1, D), Wh, Wr, Wt,
                  lp_b1.reshape(1, D), lp_W2.reshape(1, D),
                  lp_b2.reshape(1, 1))


# speedup vs baseline: 3.3089x; 1.0058x over previous
import functools

import jax
import jax.numpy as jnp
from jax import lax
from jax.experimental import pallas as pl
from jax.experimental.pallas import tpu as pltpu
from jax.experimental.pallas import tpu_sc as plsc

N = 10000
E = 320000
D = 128
B = 8192
NC = 2
NS = 16
CH = 125
NCHUNK = 80
RPT = 624
RPT_LAST = N - (NS - 1) * RPT
ZR = 16
BPT = B // (NC * NS)
GCH = 128

_sc_mesh = plsc.VectorSubcoreMesh(
    core_axis_name="c", subcore_axis_name="s", num_cores=NC, num_subcores=NS
)


@functools.partial(
    pl.kernel,
    out_type=jax.ShapeDtypeStruct((NC, N, D), jnp.float32),
    mesh=_sc_mesh,
    scratch_types=[
        pltpu.VMEM((NCHUNK // 2, CH), jnp.int32),
        pltpu.VMEM((NCHUNK // 2, CH), jnp.int32),
        pltpu.VMEM((2, CH, D), jnp.float32),
        pltpu.VMEM((ZR, D), jnp.float32),
        pltpu.VMEM_SHARED((N, D), jnp.float32),
        pltpu.SemaphoreType.DMA,
    ],
)
def _neighbor_sum(x_hbm, row2_hbm, col2_hbm, out_hbm,
                  row_v, col_v, rows_v, zb, nb_s, sem):
    c = lax.axis_index("c")
    s = lax.axis_index("s")
    tg = c * NS + s

    zeros16 = jnp.zeros((16,), jnp.float32)
    for i in range(ZR):
        for k in range(D // 16):
            zb[i, pl.ds(k * 16, 16)] = zeros16
    for q in range(RPT // ZR):
        pltpu.sync_copy(zb, nb_s.at[pl.ds(s * RPT + q * ZR, ZR)])

    @pl.when(s == NS - 1)
    def _zero_tail():
        pltpu.sync_copy(zb, nb_s.at[pl.ds(NS * RPT, N - NS * RPT)])

    plsc.subcore_barrier()

    HN = NCHUNK // 2
    for h in range(2):
        hbase = tg * NCHUNK + h * HN
        pltpu.sync_copy(row2_hbm.at[pl.ds(hbase, HN)], row_v)
        pltpu.sync_copy(col2_hbm.at[pl.ds(hbase, HN)], col_v)
        pltpu.async_copy(x_hbm.at[col_v.at[0]], rows_v.at[0], sem).wait()

        def body(j, carry):
            b = j % 2

            @pl.when(j + 1 < HN)
            def _steady():
                d = pltpu.async_copy(x_hbm.at[col_v.at[j + 1]],
                                     rows_v.at[1 - b], sem)
                pltpu.sync_copy(rows_v.at[b], nb_s.at[row_v.at[j]], add=True)
                d.wait()

            @pl.when(j + 1 >= HN)
            def _last():
                pltpu.sync_copy(rows_v.at[b], nb_s.at[row_v.at[j]], add=True)
            return carry

        lax.fori_loop(0, HN, body, 0)

    plsc.subcore_barrier()

    pltpu.sync_copy(nb_s.at[pl.ds(s * RPT, RPT)],
                    out_hbm.at[c, pl.ds(s * RPT, RPT)])

    @pl.when(s == NS - 1)
    def _write_tail():
        pltpu.sync_copy(nb_s.at[pl.ds(NS * RPT, N - NS * RPT)],
                        out_hbm.at[c, pl.ds(NS * RPT, N - NS * RPT)])


@functools.partial(
    pl.kernel,
    out_type=[jax.ShapeDtypeStruct((B, D), jnp.float32)] * 7,
    mesh=_sc_mesh,
    scratch_types=[
        pltpu.VMEM((3, GCH), jnp.int32),
        pltpu.VMEM((2, GCH, D), jnp.float32),
        pltpu.SemaphoreType.DMA,
    ],
)
def _gather7(x1, x2, x3, re_hbm, hid, tid, rid,
             oh1, oh2, oh3, ot1, ot2, ot3, org,
             idx_v, buf, sem):
    c = lax.axis_index("c")
    s = lax.axis_index("s")
    w = s * NC + c

    for k in range(BPT // GCH):
        base = w * BPT + k * GCH
        pltpu.sync_copy(hid.at[pl.ds(base, GCH)], idx_v.at[0])
        pltpu.sync_copy(tid.at[pl.ds(base, GCH)], idx_v.at[1])
        pltpu.sync_copy(rid.at[pl.ds(base, GCH)], idx_v.at[2])
        work = [(x1, 0, oh1), (x2, 0, oh2), (x3, 0, oh3),
                (x1, 1, ot1), (x2, 1, ot2), (x3, 1, ot3),
                (re_hbm, 2, org)]
        src0, i0, _ = work[0]
        d = pltpu.async_copy(src0.at[idx_v.at[i0]], buf.at[0], sem)
        for t in range(1, len(work)):
            d.wait()
            src, i, _ = work[t]
            d = pltpu.async_copy(src.at[idx_v.at[i]], buf.at[t % 2], sem)
            pltpu.sync_copy(buf.at[(t - 1) % 2],
                            work[t - 1][2].at[pl.ds(base, GCH)])
        d.wait()
        pltpu.sync_copy(buf.at[(len(work) - 1) % 2],
                        work[-1][2].at[pl.ds(base, GCH)])


BR = 1000
NB = N // BR


def _mlp_body(x_ref, nb_ref, W1_ref, b1_ref, g_ref, be_ref, W2_ref, b2_ref,
              eps_ref, out_ref):
    dot = lambda a, b: jnp.dot(a, b, preferred_element_type=jnp.float32)
    h0 = x_ref[...] * eps_ref[0, 0] + nb_ref[0] + nb_ref[1]
    hb = dot(h0, W1_ref[...]) + b1_ref[...]
    mu = jnp.mean(hb, axis=0, keepdims=True)
    dm = hb - mu
    var = jnp.mean(dm * dm, axis=0, keepdims=True)
    hn = dm * lax.rsqrt(var + 1e-5) * g_ref[...] + be_ref[...]
    out_ref[...] = dot(jnp.maximum(hn, 0.0), W2_ref[...]) + b2_ref[...]


def _mlp(x, nb, W1, b1, g, be, W2, b2, opeps):
    vm = pl.BlockSpec(memory_space=pltpu.VMEM)
    return pl.pallas_call(
        _mlp_body,
        in_specs=[vm] * 8 + [pl.BlockSpec(memory_space=pltpu.SMEM)],
        out_specs=vm,
        out_shape=jax.ShapeDtypeStruct((N, D), jnp.float32),
    )(x, nb, W1, b1, g, be, W2, b2, opeps)


SB = 1024
NSB = B // SB


def _score_body(h1, h2, h3, t1, t2, t3, rg, J1, J2, J3, jkb, Wh, Wr, Wt,
                b1r, w2r, b2s, out_ref):
    dot = lambda a, b: jnp.dot(a, b, preferred_element_type=jnp.float32)
    A1 = dot(J1[...], Wh[...])
    A2 = dot(J2[...], Wh[...])
    A3 = dot(J3[...], Wh[...])
    C1 = dot(J1[...], Wt[...])
    C2 = dot(J2[...], Wt[...])
    C3 = dot(J3[...], Wt[...])
    bias = dot(jkb[...], Wh[...]) + dot(jkb[...], Wt[...]) + b1r[...]
    pre = (dot(h1[...], A1) + dot(h2[...], A2) + dot(h3[...], A3)
           + dot(t1[...], C1) + dot(t2[...], C2) + dot(t3[...], C3)
           + dot(rg[...], Wr[...]) + bias)
    hr = jnp.maximum(pre, 0.0)
    out_ref[...] = jnp.sum(hr * w2r[...], axis=1) + b2s[0, 0]


def _score(h1, h2, h3, t1, t2, t3, rg, J1, J2, J3, jkb, Wh, Wr, Wt,
           b1r, w2r, b2s):
    blk = pl.BlockSpec((SB, D), lambda i: (i, 0))
    full = lambda shape: pl.BlockSpec(shape, lambda i: (0,) * len(shape))
    return pl.pallas_call(
        _score_body,
        grid=(NSB,),
        in_specs=[blk] * 7 + [full((D, D))] * 3 + [full((1, D))]
        + [full((D, D))] * 3 + [full((1, D)), full((1, D))]
        + [pl.BlockSpec(memory_space=pltpu.SMEM)],
        out_specs=pl.BlockSpec((SB,), lambda i: (i,)),
        out_shape=jax.ShapeDtypeStruct((B,), jnp.float32),
    )(h1, h2, h3, t1, t2, t3, rg, J1, J2, J3, jkb, Wh, Wr, Wt, b1r, w2r, b2s)


def kernel(edge_index, head_ids, relation_ids, tail_ids, ent_emb, rel_emb,
           W1_0, b1_0, g_0, be_0, W2_0, b2_0, eps_0,
           W1_1, b1_1, g_1, be_1, W2_1, b2_1, eps_1,
           W1_2, b1_2, g_2, be_2, W2_2, b2_2, eps_2,
           jk_W, jk_b, lp_W1, lp_b1, lp_W2, lp_b2):
    row2 = edge_index[0].reshape(E // CH, CH)
    col2 = edge_index[1].reshape(E // CH, CH)

    params = [
        (W1_0, b1_0, g_0, be_0, W2_0, b2_0, eps_0),
        (W1_1, b1_1, g_1, be_1, W2_1, b2_1, eps_1),
        (W1_2, b1_2, g_2, be_2, W2_2, b2_2, eps_2),
    ]
    x = ent_emb
    outs = []
    for (W1, b1, g, be, W2, b2, eps) in params:
        nb = _neighbor_sum(x, row2, col2)
        x = _mlp(x, nb, W1, b1.reshape(1, D), g.reshape(1, D),
                 be.reshape(1, D), W2, b2.reshape(1, D),
                 (1.0 + eps).reshape(1, 1))
        outs.append(x)

    x1, x2, x3 = outs
    h1, h2, h3, t1, t2, t3, rg = _gather7(
        x1, x2, x3, rel_emb, head_ids, tail_ids, relation_ids)

    J1, J2, J3 = jk_W[0:D], jk_W[D:2 * D], jk_W[2 * D:3 * D]
    Wh, Wr, Wt = lp_W1[0:D], lp_W1[D:2 * D], lp_W1[2 * D:3 * D]
    return _score(h1, h2, h3, t1, t2, t3, rg, J1, J2, J3,
                  jk_b.reshape(1, D), Wh, Wr, Wt,
                  lp_b1.reshape(1, D), lp_W2.reshape(1, D),
                  lp_b2.reshape(1, 1))
